# Initial kernel scaffold; baseline (speedup 1.0000x reference)
import jax
import jax.numpy as jnp
from jax.experimental import pallas as pl


def kernel(x, ptrs, csr):
    raise NotImplementedError("write your pallas kernel here")


# speedup vs baseline: 181.8661x; 181.8661x over previous
import functools
import jax
import jax.numpy as jnp
from jax import lax
from jax.experimental import pallas as pl
from jax.experimental.pallas import tpu as pltpu
from jax.experimental.pallas import tpu_sc as plsc

N_NODES = 100000
N_EDGES = 6400000
NC = 2
NS = 16
NW = NC * NS
EPT = N_EDGES // NW
BLK = 10000
NB = EPT // BLK
NPAD = 102400
ZB = NPAD // NS
XPAD = 100352


def _sc_body(x_hbm, ptrs_hbm, csr_hbm, out_hbm,
             xs, acc, ptr_v, csr_v, val_v, zb):
    c = lax.axis_index("c")
    s = lax.axis_index("s")

    def zfill(j, carry):
        zb[pl.ds(j * 16, 16)] = jnp.zeros((16,), jnp.float32)
        return carry
    lax.fori_loop(0, ZB // 16, zfill, 0)
    pltpu.sync_copy(zb, acc.at[pl.ds(s * ZB, ZB)])

    @pl.when(s == 0)
    def _():
        pltpu.sync_copy(x_hbm, xs)

    plsc.subcore_barrier()

    base = (c * NS + s) * EPT

    def blk(b, carry):
        off = base + b * BLK
        pltpu.sync_copy(ptrs_hbm.at[pl.ds(off, BLK)], ptr_v)
        pltpu.sync_copy(csr_hbm.at[pl.ds(off, BLK)], csr_v)
        pltpu.sync_copy(xs.at[ptr_v], val_v)
        pltpu.sync_copy(val_v, acc.at[csr_v], add=True)
        return carry
    lax.fori_loop(0, NB, blk, 0)

    plsc.subcore_barrier()

    pltpu.sync_copy(acc.at[pl.ds(s * ZB, ZB)],
                    out_hbm.at[c, pl.ds(s * ZB, ZB)])


@jax.jit
def _sc_scatter(x, ptrs, csr):
    mesh = plsc.VectorSubcoreMesh(core_axis_name="c", subcore_axis_name="s")
    f = pl.kernel(
        _sc_body, mesh=mesh,
        out_type=jax.ShapeDtypeStruct((NC, NPAD), jnp.float32),
        scratch_types=[
            pltpu.MemorySpace.VMEM_SHARED((N_NODES,), jnp.float32),
            pltpu.MemorySpace.VMEM_SHARED((NPAD,), jnp.float32),
            pltpu.VMEM((BLK,), jnp.int32),
            pltpu.VMEM((BLK,), jnp.int32),
            pltpu.VMEM((BLK,), jnp.float32),
            pltpu.VMEM((ZB,), jnp.float32),
        ],
    )
    return f(x, ptrs, csr)


def _combine_body(p_ref, o_ref):
    o_ref[...] = p_ref[0] + p_ref[1]


@jax.jit
def _combine(partials):
    p = partials.reshape(NC, NPAD // 128, 128)
    out = pl.pallas_call(
        _combine_body,
        out_shape=jax.ShapeDtypeStruct((NPAD // 128, 128), jnp.float32),
    )(p)
    return out.reshape(-1)[:N_NODES]


def kernel(x, ptrs, csr):
    partials = _sc_scatter(x, ptrs, csr)
    return _combine(partials)


# speedup vs baseline: 194.5886x; 1.0700x over previous
import jax
import jax.numpy as jnp
from jax import lax
from jax.experimental import pallas as pl
from jax.experimental.pallas import tpu as pltpu
from jax.experimental.pallas import tpu_sc as plsc

N_NODES = 100000
N_EDGES = 6400000
NC = 2
NS = 16
NW = NC * NS
EPT = N_EDGES // NW
BLK = 10000
NB = EPT // BLK
NPAD = 102400
ZB = NPAD // NS


def _sc_body(x_hbm, ptrs_hbm, csr_hbm, out_hbm,
             xs, acc,
             ptr0, ptr1, csr0, csr1, val_v, zb,
             lds0, lds1):
    c = lax.axis_index("c")
    s = lax.axis_index("s")
    ptr_v = (ptr0, ptr1)
    csr_v = (csr0, csr1)
    lds = (lds0, lds1)

    def zfill(j, carry):
        zb[pl.ds(j * 16, 16)] = jnp.zeros((16,), jnp.float32)
        return carry
    lax.fori_loop(0, ZB // 16, zfill, 0)
    pltpu.sync_copy(zb, acc.at[pl.ds(s * ZB, ZB)])

    @pl.when(s == 0)
    def _():
        pltpu.sync_copy(x_hbm, xs)

    plsc.subcore_barrier()

    base = (c * NS + s) * EPT

    def issue_loads(b, par):
        off = base + b * BLK
        pltpu.async_copy(ptrs_hbm.at[pl.ds(off, BLK)], ptr_v[par], lds[par])
        pltpu.async_copy(csr_hbm.at[pl.ds(off, BLK)], csr_v[par], lds[par])

    def wait_loads(b, par):
        off = base + b * BLK
        pltpu.make_async_copy(
            ptrs_hbm.at[pl.ds(off, BLK)], ptr_v[par], lds[par]).wait()
        pltpu.make_async_copy(
            csr_hbm.at[pl.ds(off, BLK)], csr_v[par], lds[par]).wait()

    issue_loads(0, 0)
    issue_loads(1, 1)

    def step(b, par):
        wait_loads(b, par)
        pltpu.sync_copy(xs.at[ptr_v[par]], val_v)
        pltpu.sync_copy(val_v, acc.at[csr_v[par]], add=True)
        @pl.when(b + 2 < NB)
        def _():
            issue_loads(b + 2, par)

    def pair(g, carry):
        step(2 * g, 0)
        step(2 * g + 1, 1)
        return carry
    lax.fori_loop(0, NB // 2, pair, 0)

    plsc.subcore_barrier()

    pltpu.sync_copy(acc.at[pl.ds(s * ZB, ZB)],
                    out_hbm.at[c, pl.ds(s * ZB, ZB)])


@jax.jit
def _sc_scatter(x, ptrs, csr):
    mesh = plsc.VectorSubcoreMesh(core_axis_name="c", subcore_axis_name="s")
    f = pl.kernel(
        _sc_body, mesh=mesh,
        out_type=jax.ShapeDtypeStruct((NC, NPAD), jnp.float32),
        scratch_types=[
            pltpu.MemorySpace.VMEM_SHARED((N_NODES,), jnp.float32),
            pltpu.MemorySpace.VMEM_SHARED((NPAD,), jnp.float32),
            pltpu.VMEM((BLK,), jnp.int32),
            pltpu.VMEM((BLK,), jnp.int32),
            pltpu.VMEM((BLK,), jnp.int32),
            pltpu.VMEM((BLK,), jnp.int32),
            pltpu.VMEM((BLK,), jnp.float32),
            pltpu.VMEM((ZB,), jnp.float32),
            pltpu.SemaphoreType.DMA,
            pltpu.SemaphoreType.DMA,
        ],
    )
    return f(x, ptrs, csr)


def _combine_body(p_ref, o_ref):
    o_ref[...] = p_ref[0] + p_ref[1]


@jax.jit
def _combine(partials):
    p = partials.reshape(NC, NPAD // 128, 128)
    out = pl.pallas_call(
        _combine_body,
        out_shape=jax.ShapeDtypeStruct((NPAD // 128, 128), jnp.float32),
    )(p)
    return out.reshape(-1)[:N_NODES]


def kernel(x, ptrs, csr):
    partials = _sc_scatter(x, ptrs, csr)
    return _combine(partials)


# speedup vs baseline: 268.3681x; 1.3792x over previous
import jax
import jax.numpy as jnp
from jax import lax
from jax.experimental import pallas as pl
from jax.experimental.pallas import tpu as pltpu
from jax.experimental.pallas import tpu_sc as plsc

N_NODES = 100000
N_EDGES = 6400000
NC = 2
NS = 16
NW = NC * NS
EPT = N_EDGES // NW
BLK = 10000
NB = EPT // BLK
NPAD = 102400
ZB = NPAD // NS


def _sc_body(x_hbm, ptrs_hbm, csr_hbm, out_hbm,
             xs, acc,
             ptr0, ptr1, csr0, csr1, val0, val1, zb,
             lds0, lds1, gsm0, gsm1):
    c = lax.axis_index("c")
    s = lax.axis_index("s")
    ptr_v = (ptr0, ptr1)
    csr_v = (csr0, csr1)
    val_v = (val0, val1)
    lds = (lds0, lds1)
    gsm = (gsm0, gsm1)

    def zfill(j, carry):
        zb[pl.ds(j * 16, 16)] = jnp.zeros((16,), jnp.float32)
        return carry
    lax.fori_loop(0, ZB // 16, zfill, 0)
    pltpu.sync_copy(zb, acc.at[pl.ds(s * ZB, ZB)])

    @pl.when(s == 0)
    def _():
        pltpu.sync_copy(x_hbm, xs)

    plsc.subcore_barrier()

    base = (c * NS + s) * EPT

    def issue_loads(b, par):
        off = base + b * BLK
        pltpu.async_copy(ptrs_hbm.at[pl.ds(off, BLK)], ptr_v[par], lds[par])
        pltpu.async_copy(csr_hbm.at[pl.ds(off, BLK)], csr_v[par], lds[par])

    def wait_loads(b, par):
        off = base + b * BLK
        pltpu.make_async_copy(
            ptrs_hbm.at[pl.ds(off, BLK)], ptr_v[par], lds[par]).wait()
        pltpu.make_async_copy(
            csr_hbm.at[pl.ds(off, BLK)], csr_v[par], lds[par]).wait()

    issue_loads(0, 0)
    issue_loads(1, 1)
    wait_loads(0, 0)
    pltpu.async_copy(xs.at[ptr_v[0]], val_v[0], gsm[0])

    def step(b, par):
        nxt = 1 - par
        pltpu.make_async_copy(xs.at[ptr_v[par]], val_v[par], gsm[par]).wait()
        @pl.when(b + 1 < NB)
        def _():
            wait_loads(b + 1, nxt)
            pltpu.async_copy(xs.at[ptr_v[nxt]], val_v[nxt], gsm[nxt])
        pltpu.sync_copy(val_v[par], acc.at[csr_v[par]], add=True)
        @pl.when(b + 2 < NB)
        def _():
            issue_loads(b + 2, par)

    def pair(g, carry):
        step(2 * g, 0)
        step(2 * g + 1, 1)
        return carry
    lax.fori_loop(0, NB // 2, pair, 0)

    plsc.subcore_barrier()

    pltpu.sync_copy(acc.at[pl.ds(s * ZB, ZB)],
                    out_hbm.at[c, pl.ds(s * ZB, ZB)])


@jax.jit
def _sc_scatter(x, ptrs, csr):
    mesh = plsc.VectorSubcoreMesh(core_axis_name="c", subcore_axis_name="s")
    f = pl.kernel(
        _sc_body, mesh=mesh,
        out_type=jax.ShapeDtypeStruct((NC, NPAD), jnp.float32),
        scratch_types=[
            pltpu.MemorySpace.VMEM_SHARED((N_NODES,), jnp.float32),
            pltpu.MemorySpace.VMEM_SHARED((NPAD,), jnp.float32),
            pltpu.VMEM((BLK,), jnp.int32),
            pltpu.VMEM((BLK,), jnp.int32),
            pltpu.VMEM((BLK,), jnp.int32),
            pltpu.VMEM((BLK,), jnp.int32),
            pltpu.VMEM((BLK,), jnp.float32),
            pltpu.VMEM((BLK,), jnp.float32),
            pltpu.VMEM((ZB,), jnp.float32),
            pltpu.SemaphoreType.DMA,
            pltpu.SemaphoreType.DMA,
            pltpu.SemaphoreType.DMA,
            pltpu.SemaphoreType.DMA,
        ],
    )
    return f(x, ptrs, csr)


def _combine_body(p_ref, o_ref):
    o_ref[...] = p_ref[0] + p_ref[1]


@jax.jit
def _combine(partials):
    p = partials.reshape(NC, NPAD // 128, 128)
    out = pl.pallas_call(
        _combine_body,
        out_shape=jax.ShapeDtypeStruct((NPAD // 128, 128), jnp.float32),
    )(p)
    return out.reshape(-1)[:N_NODES]


def kernel(x, ptrs, csr):
    partials = _sc_scatter(x, ptrs, csr)
    return _combine(partials)
